# baseline (speedup 1.0000x reference)
import functools

import jax
import jax.numpy as jnp
from jax import lax
from jax.experimental import pallas as pl
from jax.experimental.pallas import tpu as pltpu
from jax.experimental.pallas import tpu_sc as plsc

_CHUNK = 128


@functools.cache
def _build_gather(B: int, V: int, D: int):
    info = plsc.get_sparse_core_info()
    NC, NS = info.num_cores, info.num_subcores
    NW = NC * NS
    assert B % (NW * _CHUNK) == 0
    b_per_w = B // NW
    mesh = plsc.VectorSubcoreMesh(core_axis_name="c", subcore_axis_name="s")

    @functools.partial(
        pl.kernel,
        mesh=mesh,
        out_type=jax.ShapeDtypeStruct((D, B), jnp.float32),
        compiler_params=pltpu.CompilerParams(
            use_tc_tiling_on_sc=False,
            needs_layout_passes=False,
            skip_device_barrier=True,
            disable_bounds_checks=True,
            disable_semaphore_checks=True,
        ),
        scratch_types=[
            pltpu.VMEM((b_per_w,), jnp.int32),
            pltpu.VMEM((D, b_per_w), jnp.float32),
            pltpu.SemaphoreType.DMA,
        ],
    )
    def gather_k(ids_hbm, dataT_hbm, outT_hbm, idx_v, colv, sem):
        wid = lax.axis_index("s") * NC + lax.axis_index("c")
        base = wid * b_per_w

        pltpu.sync_copy(ids_hbm.at[pl.ds(base, b_per_w)], idx_v)
        copies = []
        for c in range(D):
            col = dataT_hbm.at[c]
            for j in range(b_per_w // _CHUNK):
                copies.append(pltpu.async_copy(
                    col.at[idx_v.at[pl.ds(j * _CHUNK, _CHUNK)]],
                    colv.at[c, pl.ds(j * _CHUNK, _CHUNK)],
                    sem,
                ))
        for cp in copies:
            cp.wait()
        for c in range(D):
            pltpu.sync_copy(
                colv.at[c], outT_hbm.at[c, pl.ds(base, b_per_w)]
            )

    return gather_k


def kernel(ids, data):
    B, = ids.shape
    V, D = data.shape
    gather_k = _build_gather(B, V, D)
    return gather_k(ids.astype(jnp.int32), data.T).T


# speedup vs baseline: 1.0021x; 1.0021x over previous
import functools

import jax
import jax.numpy as jnp
from jax import lax
from jax.experimental import pallas as pl
from jax.experimental.pallas import tpu as pltpu
from jax.experimental.pallas import tpu_sc as plsc

_CHUNK = 128


@functools.cache
def _build_gather(B: int, V: int, D: int):
    info = plsc.get_sparse_core_info()
    NC, NS = info.num_cores, info.num_subcores
    NW = NC * NS
    assert B % (NW * _CHUNK) == 0
    b_per_w = B // NW
    mesh = plsc.VectorSubcoreMesh(core_axis_name="c", subcore_axis_name="s")

    @functools.partial(
        pl.kernel,
        mesh=mesh,
        out_type=jax.ShapeDtypeStruct((D, B), jnp.float32),
        compiler_params=pltpu.CompilerParams(
            use_tc_tiling_on_sc=False, needs_layout_passes=False
        ),
        scratch_types=[
            pltpu.VMEM((b_per_w,), jnp.int32),
            pltpu.VMEM((D, b_per_w), jnp.float32),
            pltpu.SemaphoreType.DMA,
        ],
    )
    def gather_k(ids_hbm, dataT_hbm, outT_hbm, idx_v, colv, sem):
        wid = lax.axis_index("s") * NC + lax.axis_index("c")
        base = wid * b_per_w

        pltpu.sync_copy(ids_hbm.at[pl.ds(base, b_per_w)], idx_v)
        copies = []
        for c in range(D):
            col = dataT_hbm.at[c]
            for j in range(b_per_w // _CHUNK):
                copies.append(pltpu.async_copy(
                    col.at[idx_v.at[pl.ds(j * _CHUNK, _CHUNK)]],
                    colv.at[c, pl.ds(j * _CHUNK, _CHUNK)],
                    sem,
                ))
        for cp in copies:
            cp.wait()
        for c in range(D):
            pltpu.sync_copy(
                colv.at[c], outT_hbm.at[c, pl.ds(base, b_per_w)]
            )

    return gather_k


def kernel(ids, data):
    B, = ids.shape
    V, D = data.shape
    gather_k = _build_gather(B, V, D)
    return gather_k(ids.astype(jnp.int32), data.T).T
